# baseline (speedup 1.0000x reference)
import functools

import jax
import jax.numpy as jnp
from jax import lax
from jax.experimental import pallas as pl
from jax.experimental.pallas import tpu as pltpu
from jax.experimental.pallas import tpu_sc as plsc

N = 262144
C = 100
NB = 15
HB = 16
HTOT = C * HB

NPIPE = 16
NH = N // NPIPE
NW = 32
ROWS_W = NH // NW
R = 128
NCHUNKS = ROWS_W // R
GROUPS = R // 16
BLK = 400
NVEC = BLK // 16
NBLK = (R * C) // BLK

BR = 2048


def _softmax_pack_body(x_ref, o_ref, cnt_ref):
    x = x_ref[...]
    m = jnp.max(x, axis=1, keepdims=True)
    e = jnp.exp(x - m)
    s = jnp.sum(e, axis=1, keepdims=True)
    cv = e * (1.0 / s)
    t = jnp.minimum((cv * float(NB)).astype(jnp.int32), NB - 1)
    u = lax.bitcast_convert_type(cv, jnp.int32)
    packed = (u & jnp.int32(~15)) | t
    o_ref[...] = lax.bitcast_convert_type(packed, jnp.float32)
    blk = jnp.concatenate(
        [jnp.sum((t == b).astype(jnp.float32), axis=0, keepdims=True)
         for b in range(NB)],
        axis=0,
    )

    @pl.when(pl.program_id(0) == 0)
    def _init():
        cnt_ref[...] = jnp.zeros((NB, C), jnp.float32)

    cnt_ref[...] += blk


def _softmax_pack(logits):
    return pl.pallas_call(
        _softmax_pack_body,
        grid=(NH // BR,),
        in_specs=[pl.BlockSpec((BR, C), lambda i: (i, 0))],
        out_specs=[
            pl.BlockSpec((BR, C), lambda i: (i, 0)),
            pl.BlockSpec((NB, C), lambda i: (0, 0)),
        ],
        out_shape=[
            jax.ShapeDtypeStruct((NH, C), jnp.float32),
            jax.ShapeDtypeStruct((NB, C), jnp.float32),
        ],
    )(logits)


def _sc_body(conf_hbm, labels_hbm, out_hbm, chunk_v, labels_v,
             conf_h, acc_h):
    wid = lax.axis_index("s") * 2 + lax.axis_index("c")
    zero16 = jnp.zeros((16,), jnp.float32)
    ones16 = jnp.ones((16,), jnp.float32)
    lane = lax.broadcasted_iota(jnp.int32, (16,), 0)
    rowoff0 = lane * C
    segbase = [((v * 16 + lane) % C) * HB for v in range(NVEC)]

    def zero_body(i, _):
        conf_h[pl.ds(i * 16, 16)] = zero16
        acc_h[pl.ds(i * 16, 16)] = zero16
        return 0
    lax.fori_loop(0, HTOT // 16, zero_body, 0)

    pltpu.sync_copy(labels_hbm.at[pl.ds(wid * ROWS_W, ROWS_W)], labels_v)

    def chunk_body(ci, _):
        row_base = wid * ROWS_W + ci * R
        pltpu.sync_copy(conf_hbm.at[pl.ds(row_base * C, R * C)], chunk_v)

        def blk_body(b, _):
            base = b * BLK
            for v in range(NVEC):
                w = chunk_v[pl.ds(base + v * 16, 16)]
                t = lax.bitcast_convert_type(w, jnp.int32) & 15
                seg = segbase[v] + t
                plsc.addupdate_scatter(conf_h, [seg], w)
            return 0
        lax.fori_loop(0, NBLK, blk_body, 0)

        def group_acc(g, _):
            ro = rowoff0 + g * (16 * C)
            lbl = labels_v[pl.ds(ci * R + g * 16, 16)]
            wv = plsc.load_gather(chunk_v, [ro + lbl])
            t = lax.bitcast_convert_type(wv, jnp.int32) & 15
            plsc.addupdate_scatter(acc_h, [lbl * HB + t], ones16)
            return 0
        lax.fori_loop(0, GROUPS, group_acc, 0)
        return 0
    lax.fori_loop(0, NCHUNKS, chunk_body, 0)

    pltpu.sync_copy(conf_h, out_hbm.at[wid])
    pltpu.sync_copy(acc_h, out_hbm.at[NW + wid])


@functools.partial(
    pl.kernel,
    out_type=jax.ShapeDtypeStruct((2 * NW, HTOT), jnp.float32),
    mesh=plsc.VectorSubcoreMesh(core_axis_name="c", subcore_axis_name="s"),
    scratch_types=[
        pltpu.VMEM((R * C,), jnp.float32),
        pltpu.VMEM((ROWS_W,), jnp.int32),
        pltpu.VMEM((HTOT,), jnp.float32),
        pltpu.VMEM((HTOT,), jnp.float32),
    ],
    compiler_params=pltpu.CompilerParams(needs_layout_passes=False),
)
def _sc_hist(conf_hbm, labels_hbm, out_hbm, *scratch):
    _sc_body(conf_hbm, labels_hbm, out_hbm, *scratch)


def _finalize_body(*refs):
    cnt_ref, h_refs, (pc_ref, sce_ref) = refs[0], refs[1:1 + NPIPE], refs[1 + NPIPE:]
    h = h_refs[0][...]
    for r in h_refs[1:]:
        h = h + r[...]
    counts = cnt_ref[...]
    confs = jnp.sum(h[0:NW], axis=0, keepdims=True)
    accs = jnp.sum(h[NW:2 * NW], axis=0, keepdims=True)
    safe = jnp.maximum(counts, 1.0)
    contrib = jnp.where(
        counts > 0.0,
        jnp.abs(confs / safe - accs / safe) * (counts * (1.0 / N)),
        0.0,
    )
    row = lax.broadcasted_iota(jnp.int32, (HTOT, C), 0)
    col = lax.broadcasted_iota(jnp.int32, (HTOT, C), 1)
    pick = (row // HB == col).astype(jnp.float32)
    pc = jnp.dot(contrib, pick, preferred_element_type=jnp.float32)
    pc_ref[...] = pc
    sce_ref[...] = jnp.sum(pc, axis=(0, 1), keepdims=True) * (1.0 / C)


def _finalize(cnt_flat, hs):
    return pl.pallas_call(
        _finalize_body,
        out_shape=[
            jax.ShapeDtypeStruct((1, C), jnp.float32),
            jax.ShapeDtypeStruct((1, 1), jnp.float32),
        ],
    )(cnt_flat, *hs)


def kernel(logits, labels):
    hs, cnts = [], []
    for k in range(NPIPE):
        conf_k, cnt_k = _softmax_pack(logits[k * NH:(k + 1) * NH])
        hs.append(_sc_hist(conf_k.reshape(NH * C), labels[k * NH:(k + 1) * NH]))
        cnts.append(cnt_k)
    cnt_total = sum(cnts)
    cnt_flat = jnp.pad(cnt_total.T, ((0, 0), (0, HB - NB))).reshape(1, HTOT)
    pc, sce = _finalize(cnt_flat, hs)
    return sce.reshape(()), pc.reshape(C)


# speedup vs baseline: 1.0146x; 1.0146x over previous
import functools

import jax
import jax.numpy as jnp
from jax import lax
from jax.experimental import pallas as pl
from jax.experimental.pallas import tpu as pltpu
from jax.experimental.pallas import tpu_sc as plsc

N = 262144
C = 100
NB = 15
HB = 16
HTOT = C * HB

NPIPE = 8
NH = N // NPIPE
NW = 32
ROWS_W = NH // NW
R = 128
NCHUNKS = ROWS_W // R
GROUPS = R // 16
BLK = 400
NVEC = BLK // 16
NBLK = (R * C) // BLK

BR = 2048


def _softmax_pack_body(x_ref, o_ref, cnt_ref):
    x = x_ref[...]
    m = jnp.max(x, axis=1, keepdims=True)
    e = jnp.exp(x - m)
    s = jnp.sum(e, axis=1, keepdims=True)
    cv = e * (1.0 / s)
    t = jnp.minimum((cv * float(NB)).astype(jnp.int32), NB - 1)
    u = lax.bitcast_convert_type(cv, jnp.int32)
    packed = (u & jnp.int32(~15)) | t
    o_ref[...] = lax.bitcast_convert_type(packed, jnp.float32)
    blk = jnp.concatenate(
        [jnp.sum((t == b).astype(jnp.float32), axis=0, keepdims=True)
         for b in range(NB)],
        axis=0,
    )

    @pl.when(pl.program_id(0) == 0)
    def _init():
        cnt_ref[...] = jnp.zeros((NB, C), jnp.float32)

    cnt_ref[...] += blk


def _softmax_pack(logits):
    return pl.pallas_call(
        _softmax_pack_body,
        grid=(NH // BR,),
        in_specs=[pl.BlockSpec((BR, C), lambda i: (i, 0))],
        out_specs=[
            pl.BlockSpec((BR, C), lambda i: (i, 0)),
            pl.BlockSpec((NB, C), lambda i: (0, 0)),
        ],
        out_shape=[
            jax.ShapeDtypeStruct((NH, C), jnp.float32),
            jax.ShapeDtypeStruct((NB, C), jnp.float32),
        ],
    )(logits)


def _sc_body(conf_hbm, labels_hbm, out_hbm, chunk_v, labels_v,
             conf_h, acc_h):
    wid = lax.axis_index("s") * 2 + lax.axis_index("c")
    zero16 = jnp.zeros((16,), jnp.float32)
    ones16 = jnp.ones((16,), jnp.float32)
    lane = lax.broadcasted_iota(jnp.int32, (16,), 0)
    rowoff0 = lane * C
    segbase = [((v * 16 + lane) % C) * HB for v in range(NVEC)]

    def zero_body(i, _):
        conf_h[pl.ds(i * 16, 16)] = zero16
        acc_h[pl.ds(i * 16, 16)] = zero16
        return 0
    lax.fori_loop(0, HTOT // 16, zero_body, 0)

    pltpu.sync_copy(labels_hbm.at[pl.ds(wid * ROWS_W, ROWS_W)], labels_v)

    def chunk_body(ci, _):
        row_base = wid * ROWS_W + ci * R
        pltpu.sync_copy(conf_hbm.at[pl.ds(row_base * C, R * C)], chunk_v)

        def blk_body(b, _):
            base = b * BLK
            for v in range(NVEC):
                w = chunk_v[pl.ds(base + v * 16, 16)]
                t = lax.bitcast_convert_type(w, jnp.int32) & 15
                seg = segbase[v] + t
                plsc.addupdate_scatter(conf_h, [seg], w)
            return 0
        lax.fori_loop(0, NBLK, blk_body, 0)

        def group_acc(g, _):
            ro = rowoff0 + g * (16 * C)
            lbl = labels_v[pl.ds(ci * R + g * 16, 16)]
            wv = plsc.load_gather(chunk_v, [ro + lbl])
            t = lax.bitcast_convert_type(wv, jnp.int32) & 15
            plsc.addupdate_scatter(acc_h, [lbl * HB + t], ones16)
            return 0
        lax.fori_loop(0, GROUPS, group_acc, 0)
        return 0
    lax.fori_loop(0, NCHUNKS, chunk_body, 0)

    pltpu.sync_copy(conf_h, out_hbm.at[wid])
    pltpu.sync_copy(acc_h, out_hbm.at[NW + wid])


@functools.partial(
    pl.kernel,
    out_type=jax.ShapeDtypeStruct((2 * NW, HTOT), jnp.float32),
    mesh=plsc.VectorSubcoreMesh(core_axis_name="c", subcore_axis_name="s"),
    scratch_types=[
        pltpu.VMEM((R * C,), jnp.float32),
        pltpu.VMEM((ROWS_W,), jnp.int32),
        pltpu.VMEM((HTOT,), jnp.float32),
        pltpu.VMEM((HTOT,), jnp.float32),
    ],
    compiler_params=pltpu.CompilerParams(needs_layout_passes=False),
)
def _sc_hist(conf_hbm, labels_hbm, out_hbm, *scratch):
    _sc_body(conf_hbm, labels_hbm, out_hbm, *scratch)


def _finalize_body(*refs):
    cnt_ref, h_refs, (pc_ref, sce_ref) = refs[0], refs[1:1 + NPIPE], refs[1 + NPIPE:]
    h = h_refs[0][...]
    for r in h_refs[1:]:
        h = h + r[...]
    counts = cnt_ref[...]
    confs = jnp.sum(h[0:NW], axis=0, keepdims=True)
    accs = jnp.sum(h[NW:2 * NW], axis=0, keepdims=True)
    safe = jnp.maximum(counts, 1.0)
    contrib = jnp.where(
        counts > 0.0,
        jnp.abs(confs / safe - accs / safe) * (counts * (1.0 / N)),
        0.0,
    )
    row = lax.broadcasted_iota(jnp.int32, (HTOT, C), 0)
    col = lax.broadcasted_iota(jnp.int32, (HTOT, C), 1)
    pick = (row // HB == col).astype(jnp.float32)
    pc = jnp.dot(contrib, pick, preferred_element_type=jnp.float32)
    pc_ref[...] = pc
    sce_ref[...] = jnp.sum(pc, axis=(0, 1), keepdims=True) * (1.0 / C)


def _finalize(cnt_flat, hs):
    return pl.pallas_call(
        _finalize_body,
        out_shape=[
            jax.ShapeDtypeStruct((1, C), jnp.float32),
            jax.ShapeDtypeStruct((1, 1), jnp.float32),
        ],
    )(cnt_flat, *hs)


def kernel(logits, labels):
    hs, cnts = [], []
    for k in range(NPIPE):
        conf_k, cnt_k = _softmax_pack(logits[k * NH:(k + 1) * NH])
        hs.append(_sc_hist(conf_k.reshape(NH * C), labels[k * NH:(k + 1) * NH]))
        cnts.append(cnt_k)
    cnt_total = sum(cnts)
    cnt_flat = jnp.pad(cnt_total.T, ((0, 0), (0, HB - NB))).reshape(1, HTOT)
    pc, sce = _finalize(cnt_flat, hs)
    return sce.reshape(()), pc.reshape(C)
